# baseline (speedup 1.0000x reference)
import jax
import jax.numpy as jnp
from jax import lax
from jax.experimental import pallas as pl
from jax.experimental.pallas import tpu as pltpu
from jax.experimental.pallas import tpu_sc as plsc

_N, _D = 32640, 512
_K = 3
_DK = _D * _K
_NW = 32
_RPW = _N // _NW
_C = 51
_NCHUNK = _RPW // _C
_CP2 = _C + 2
_INROWS = _CP2 + 2
_L = 16
_NG = _DK // _L


def _sc_body(x_hbm, csum_hbm, out_hbm,
             in_v, out_v, csum_v, mp_v, mn_v, fidx_v):
    wid = lax.axis_index("s") * 2 + lax.axis_index("c")
    base = wid * _RPW

    pltpu.sync_copy(csum_hbm, csum_v)

    ones = jnp.ones((_L,), jnp.float32)
    zeros = jnp.zeros((_L,), jnp.float32)
    lane = lax.broadcasted_iota(jnp.int32, (_L,), 0)

    def init_flags(k, c):
        mp_v[pl.ds(k * _L, _L)] = ones
        mn_v[pl.ds(k * _L, _L)] = ones
        return c
    lax.fori_loop(0, (_RPW + 2 * _L) // _L, init_flags, 0)

    for u in range(_NG):
        t = lane + u * _L
        j = lax.shift_right_logical(t * 21846, 16)
        fidx_v[pl.ds(u * _L, _L)] = (t - _K * j) * _D + j

    def bflags(k, c):
        cs = csum_v[pl.ds(k * _L, _L)]
        loc = cs - base
        okp = (loc >= 0) & (loc < _RPW)
        plsc.store_scatter(mp_v, [jnp.clip(loc, 0, _RPW - 1)], zeros, mask=okp)
        loce = loc - 1
        oke = (loce >= 0) & (loce < _RPW)
        plsc.store_scatter(mn_v, [jnp.clip(loce, 0, _RPW - 1)], zeros, mask=oke)
        return c
    lax.fori_loop(0, 256 // _L, bflags, 0)

    @pl.when(wid == 0)
    def _():
        plsc.store_scatter(mp_v, [lane], zeros, mask=(lane == 0))

        def zhalo(v, c):
            in_v[pl.ds(_D + v * _L, _L)] = zeros
            return c
        lax.fori_loop(0, _D // _L, zhalo, 0)

    def chunk(q, c):
        s = base + q * _C
        src_lo = jnp.clip(s - 1, 0, _N - _CP2)
        dst_lo = 1 + (src_lo - (s - 1))
        pltpu.sync_copy(x_hbm.at[pl.ds(src_lo * _D, _CP2 * _D)],
                        in_v.at[pl.ds(dst_lo * _D, _CP2 * _D)])

        def grpu(u, cc):
            ob = u * _L
            fidx = fidx_v[pl.ds(ob, _L)]
            for r in range(_C):
                win = in_v.at[pl.ds((r + 1) * _D, _K * _D)]
                g = plsc.load_gather(win, [fidx])
                out_v[pl.ds(ob + r * _DK, _L)] = g
            return cc
        lax.fori_loop(0, _NG, grpu, 0)

        def fixr(r, cc):
            lr = q * _C + r
            base3 = lane * _K + r * _DK

            @pl.when(mp_v[pl.ds(lr, _L)][0] == 0.0)
            def _():
                def fz(v, c4):
                    plsc.store_scatter(out_v, [base3 + v * (_K * _L)], zeros)
                    return c4
                lax.fori_loop(0, _D // _L, fz, 0)

            @pl.when(mn_v[pl.ds(lr, _L)][0] == 0.0)
            def _():
                def fz(v, c4):
                    plsc.store_scatter(out_v, [base3 + v * (_K * _L) + 2],
                                       zeros)
                    return c4
                lax.fori_loop(0, _D // _L, fz, 0)
            return cc
        lax.fori_loop(0, _C, fixr, 0)

        pltpu.sync_copy(out_v, out_hbm.at[pl.ds(s * _DK, _C * _DK)])
        return c
    lax.fori_loop(0, _NCHUNK, chunk, 0)


def kernel(x, lengths):
    csum = jnp.cumsum(lengths.astype(jnp.int32))
    mesh = plsc.VectorSubcoreMesh(core_axis_name="c", subcore_axis_name="s")
    run = pl.kernel(
        _sc_body,
        mesh=mesh,
        compiler_params=pltpu.CompilerParams(needs_layout_passes=False),
        out_type=jax.ShapeDtypeStruct((_N * _DK,), jnp.float32),
        scratch_types=[
            pltpu.VMEM((_INROWS * _D,), jnp.float32),
            pltpu.VMEM((_C * _DK,), jnp.float32),
            pltpu.VMEM((256,), jnp.int32),
            pltpu.VMEM((_RPW + 2 * _L,), jnp.float32),
            pltpu.VMEM((_RPW + 2 * _L,), jnp.float32),
            pltpu.VMEM((_NG * _L,), jnp.int32),
        ],
    )
    return run(x.reshape(-1), csum).reshape(_N, _DK)


# speedup vs baseline: 1.7304x; 1.7304x over previous
import jax
import jax.numpy as jnp
from jax import lax
from jax.experimental import pallas as pl
from jax.experimental.pallas import tpu as pltpu
from jax.experimental.pallas import tpu_sc as plsc

_N, _D = 32640, 512
_K = 3
_DK = _D * _K
_NW = 32
_RPW = _N // _NW
_C = 51
_NCHUNK = _RPW // _C
_CP2 = _C + 2
_INROWS = _CP2 + 2
_L = 16
_NG = _DK // _L


def _sc_body(x_hbm, csum_hbm, out_hbm,
             in_v, out_v, csum_v, mp_v, mn_v, fidx_v):
    wid = lax.axis_index("s") * 2 + lax.axis_index("c")
    base = wid * _RPW

    pltpu.sync_copy(csum_hbm, csum_v)

    ones = jnp.ones((_L,), jnp.float32)
    zeros = jnp.zeros((_L,), jnp.float32)
    lane = lax.broadcasted_iota(jnp.int32, (_L,), 0)

    def init_flags(k, c):
        mp_v[pl.ds(k * _L, _L)] = ones
        mn_v[pl.ds(k * _L, _L)] = ones
        return c
    lax.fori_loop(0, (_RPW + 2 * _L) // _L, init_flags, 0)

    for u in range(_NG):
        t = lane + u * _L
        j = lax.shift_right_logical(t * 21846, 16)
        fidx_v[pl.ds(u * _L, _L)] = (t - _K * j) * _D + j

    def bflags(k, c):
        cs = csum_v[pl.ds(k * _L, _L)]
        loc = cs - base
        okp = (loc >= 0) & (loc < _RPW)
        plsc.store_scatter(mp_v, [jnp.clip(loc, 0, _RPW - 1)], zeros, mask=okp)
        loce = loc - 1
        oke = (loce >= 0) & (loce < _RPW)
        plsc.store_scatter(mn_v, [jnp.clip(loce, 0, _RPW - 1)], zeros, mask=oke)
        return c
    lax.fori_loop(0, 256 // _L, bflags, 0)

    @pl.when(wid == 0)
    def _():
        plsc.store_scatter(mp_v, [lane], zeros, mask=(lane == 0))

        def zhalo(v, c):
            in_v[pl.ds(_D + v * _L, _L)] = zeros
            return c
        lax.fori_loop(0, _D // _L, zhalo, 0)

    def chunk(q, c):
        s = base + q * _C
        src_lo = jnp.clip(s - 1, 0, _N - _CP2)
        dst_lo = 1 + (src_lo - (s - 1))
        pltpu.sync_copy(x_hbm.at[pl.ds(src_lo * _D, _CP2 * _D)],
                        in_v.at[pl.ds(dst_lo * _D, _CP2 * _D)])

        @plsc.parallel_loop(0, _NG, 1, unroll=2)
        def _(u):
            ob = u * _L
            fidx = fidx_v[pl.ds(ob, _L)]
            for r in range(_C):
                win = in_v.at[pl.ds((r + 1) * _D, _K * _D)]
                g = plsc.load_gather(win, [fidx])
                out_v[pl.ds(ob + r * _DK, _L)] = g

        def fixr(r, cc):
            lr = q * _C + r
            base3 = lane * _K + r * _DK

            @pl.when(mp_v[pl.ds(lr, _L)][0] == 0.0)
            def _():
                def fz(v, c4):
                    plsc.store_scatter(out_v, [base3 + v * (_K * _L)], zeros)
                    return c4
                lax.fori_loop(0, _D // _L, fz, 0)

            @pl.when(mn_v[pl.ds(lr, _L)][0] == 0.0)
            def _():
                def fz(v, c4):
                    plsc.store_scatter(out_v, [base3 + v * (_K * _L) + 2],
                                       zeros)
                    return c4
                lax.fori_loop(0, _D // _L, fz, 0)
            return cc
        lax.fori_loop(0, _C, fixr, 0)

        pltpu.sync_copy(out_v, out_hbm.at[pl.ds(s * _DK, _C * _DK)])
        return c
    lax.fori_loop(0, _NCHUNK, chunk, 0)


def kernel(x, lengths):
    csum = jnp.cumsum(lengths.astype(jnp.int32))
    mesh = plsc.VectorSubcoreMesh(core_axis_name="c", subcore_axis_name="s")
    run = pl.kernel(
        _sc_body,
        mesh=mesh,
        compiler_params=pltpu.CompilerParams(needs_layout_passes=False),
        out_type=jax.ShapeDtypeStruct((_N * _DK,), jnp.float32),
        scratch_types=[
            pltpu.VMEM((_INROWS * _D,), jnp.float32),
            pltpu.VMEM((_C * _DK,), jnp.float32),
            pltpu.VMEM((256,), jnp.int32),
            pltpu.VMEM((_RPW + 2 * _L,), jnp.float32),
            pltpu.VMEM((_RPW + 2 * _L,), jnp.float32),
            pltpu.VMEM((_NG * _L,), jnp.int32),
        ],
    )
    return run(x.reshape(-1), csum).reshape(_N, _DK)


# speedup vs baseline: 1.8222x; 1.0531x over previous
import jax
import jax.numpy as jnp
from jax import lax
from jax.experimental import pallas as pl
from jax.experimental.pallas import tpu as pltpu
from jax.experimental.pallas import tpu_sc as plsc

_N, _D = 32640, 512
_K = 3
_DK = _D * _K
_NW = 32
_RPW = _N // _NW
_C = 20
_NCHUNK = _RPW // _C
_CP2 = _C + 2
_INROWS = _CP2 + 2
_L = 16
_NG = _DK // _L


def _sc_body(x_hbm, csum_hbm, out_hbm,
             in_v, out_a, out_b, csum_v, mp_v, mn_v, fidx_v, sem_a, sem_b):
    wid = lax.axis_index("s") * 2 + lax.axis_index("c")
    base = wid * _RPW

    pltpu.sync_copy(csum_hbm, csum_v)

    ones = jnp.ones((_L,), jnp.float32)
    zeros = jnp.zeros((_L,), jnp.float32)
    lane = lax.broadcasted_iota(jnp.int32, (_L,), 0)

    def init_flags(k, c):
        mp_v[pl.ds(k * _L, _L)] = ones
        mn_v[pl.ds(k * _L, _L)] = ones
        return c
    lax.fori_loop(0, (_RPW + 2 * _L) // _L, init_flags, 0)

    for u in range(_NG):
        t = lane + u * _L
        j = lax.shift_right_logical(t * 21846, 16)
        fidx_v[pl.ds(u * _L, _L)] = (t - _K * j) * _D + j

    def bflags(k, c):
        cs = csum_v[pl.ds(k * _L, _L)]
        loc = cs - base
        okp = (loc >= 0) & (loc < _RPW)
        plsc.store_scatter(mp_v, [jnp.clip(loc, 0, _RPW - 1)], zeros, mask=okp)
        loce = loc - 1
        oke = (loce >= 0) & (loce < _RPW)
        plsc.store_scatter(mn_v, [jnp.clip(loce, 0, _RPW - 1)], zeros, mask=oke)
        return c
    lax.fori_loop(0, 256 // _L, bflags, 0)

    @pl.when(wid == 0)
    def _():
        plsc.store_scatter(mp_v, [lane], zeros, mask=(lane == 0))

        def zhalo(v, c):
            in_v[pl.ds(_D + v * _L, _L)] = zeros
            return c
        lax.fori_loop(0, _D // _L, zhalo, 0)

    def process(q, out_v, sem, drain):
        s = base + q * _C
        src_lo = jnp.clip(s - 1, 0, _N - _CP2)
        dst_lo = 1 + (src_lo - (s - 1))
        pltpu.sync_copy(x_hbm.at[pl.ds(src_lo * _D, _CP2 * _D)],
                        in_v.at[pl.ds(dst_lo * _D, _CP2 * _D)])

        @pl.when(drain)
        def _():
            pltpu.make_async_copy(
                out_v, out_hbm.at[pl.ds((s - 2 * _C) * _DK, _C * _DK)],
                sem).wait()

        @plsc.parallel_loop(0, _NG, 1, unroll=2)
        def _(u):
            ob = u * _L
            fidx = fidx_v[pl.ds(ob, _L)]
            for r in range(_C):
                win = in_v.at[pl.ds((r + 1) * _D, _K * _D)]
                g = plsc.load_gather(win, [fidx])
                out_v[pl.ds(ob + r * _DK, _L)] = g

        def fixr(r, cc):
            lr = q * _C + r
            base3 = lane * _K + r * _DK

            @pl.when(mp_v[pl.ds(lr, _L)][0] == 0.0)
            def _():
                def fz(v, c4):
                    plsc.store_scatter(out_v, [base3 + v * (_K * _L)], zeros)
                    return c4
                lax.fori_loop(0, _D // _L, fz, 0)

            @pl.when(mn_v[pl.ds(lr, _L)][0] == 0.0)
            def _():
                def fz(v, c4):
                    plsc.store_scatter(out_v, [base3 + v * (_K * _L) + 2],
                                       zeros)
                    return c4
                lax.fori_loop(0, _D // _L, fz, 0)
            return cc
        lax.fori_loop(0, _C, fixr, 0)

        pltpu.async_copy(out_v, out_hbm.at[pl.ds(s * _DK, _C * _DK)], sem)

    def pair(t, c):
        process(2 * t, out_a, sem_a, t > 0)
        process(2 * t + 1, out_b, sem_b, t > 0)
        return c
    lax.fori_loop(0, (_NCHUNK - 1) // 2, pair, 0)

    last = _NCHUNK - 1
    process(last, out_a, sem_a, True)
    s_last = base + last * _C
    pltpu.make_async_copy(out_a, out_hbm.at[pl.ds(s_last * _DK, _C * _DK)],
                          sem_a).wait()
    pltpu.make_async_copy(
        out_b, out_hbm.at[pl.ds((s_last - 1) * _DK, _C * _DK)], sem_b).wait()


def kernel(x, lengths):
    csum = jnp.cumsum(lengths.astype(jnp.int32))
    mesh = plsc.VectorSubcoreMesh(core_axis_name="c", subcore_axis_name="s")
    run = pl.kernel(
        _sc_body,
        mesh=mesh,
        compiler_params=pltpu.CompilerParams(needs_layout_passes=False),
        out_type=jax.ShapeDtypeStruct((_N * _DK,), jnp.float32),
        scratch_types=[
            pltpu.VMEM((_INROWS * _D,), jnp.float32),
            pltpu.VMEM((_C * _DK,), jnp.float32),
            pltpu.VMEM((_C * _DK,), jnp.float32),
            pltpu.VMEM((256,), jnp.int32),
            pltpu.VMEM((_RPW + 2 * _L,), jnp.float32),
            pltpu.VMEM((_RPW + 2 * _L,), jnp.float32),
            pltpu.VMEM((_NG * _L,), jnp.int32),
            pltpu.SemaphoreType.DMA,
            pltpu.SemaphoreType.DMA,
        ],
    )
    return run(x.reshape(-1), csum).reshape(_N, _DK)


# speedup vs baseline: 2.2119x; 1.2139x over previous
import jax
import jax.numpy as jnp
from jax import lax
from jax.experimental import pallas as pl
from jax.experimental.pallas import tpu as pltpu
from jax.experimental.pallas import tpu_sc as plsc

_N, _D = 32640, 512
_K = 3
_DK = _D * _K
_NW = 32
_C = 32
_NCH = _N // _C
_QMAX = (_NCH + _NW - 1) // _NW
_WIN = 64
_INROWS = 96
_L = 16
_NG = _DK // _L
_FMAX = _QMAX * _C


def _sc_body(x_hbm, csum_hbm, out_hbm,
             in_v, out_v, csum_v, mp_v, mn_v, rowm_v, col_v):
    wid = lax.axis_index("s") * 2 + lax.axis_index("c")

    pltpu.sync_copy(csum_hbm, csum_v)

    ones = jnp.ones((_L,), jnp.float32)
    zeros = jnp.zeros((_L,), jnp.float32)
    lane = lax.broadcasted_iota(jnp.int32, (_L,), 0)

    def init_flags(k, c):
        mp_v[pl.ds(k * _L, _L)] = ones
        mn_v[pl.ds(k * _L, _L)] = ones
        return c
    lax.fori_loop(0, (_FMAX + 2 * _L) // _L, init_flags, 0)

    for u in range(_NG):
        t = lane + u * _L
        j = lax.shift_right_logical(t * 21846, 16)
        rowm_v[pl.ds(u * _L, _L)] = t - _K * j
        col_v[pl.ds(u * _L, _L)] = j

    def bflags(k, c):
        cs = csum_v[pl.ds(k * _L, _L)]
        cid = lax.shift_right_logical(cs, 5)
        wv = cid & (_NW - 1)
        locs = (lax.shift_right_logical(cid, 5) * _C) + (cs & (_C - 1))
        okp = (wv == wid) & (cs < _N)
        plsc.store_scatter(mp_v, [jnp.clip(locs, 0, _FMAX - 1)], zeros,
                           mask=okp)
        ce = cs - 1
        cide = lax.shift_right_logical(ce, 5)
        wve = cide & (_NW - 1)
        loce = (lax.shift_right_logical(cide, 5) * _C) + (ce & (_C - 1))
        oke = (wve == wid) & (ce >= 0)
        plsc.store_scatter(mn_v, [jnp.clip(loce, 0, _FMAX - 1)], zeros,
                           mask=oke)
        return c
    lax.fori_loop(0, 256 // _L, bflags, 0)

    @pl.when(wid == 0)
    def _():
        plsc.store_scatter(mp_v, [lane], zeros, mask=(lane == 0))

        def zhalo(v, c):
            in_v[31, pl.ds(v * _L, _L)] = zeros
            return c
        lax.fori_loop(0, _D // _L, zhalo, 0)

    def chunk(q, c):
        cid = wid + _NW * q

        @pl.when(cid < _NCH)
        def _():
            s = cid * _C
            src_lo = pl.multiple_of(jnp.clip(s - 16, 0, _N - _WIN), 8)
            dst_lo = pl.multiple_of(src_lo - s + 32, 8)
            pltpu.sync_copy(x_hbm.at[pl.ds(src_lo, _WIN)],
                            in_v.at[pl.ds(dst_lo, _WIN)])

            @plsc.parallel_loop(0, _NG, 1, unroll=2)
            def _(u):
                ob = u * _L
                rowm = rowm_v[pl.ds(ob, _L)]
                col = col_v[pl.ds(ob, _L)]
                for r in range(_C):
                    g = plsc.load_gather(in_v, [rowm + (r + 31), col])
                    out_v[r, pl.ds(ob, _L)] = g

            def fixr(r, cc):
                lr = q * _C + r
                base3 = lane * _K

                @pl.when(mp_v[pl.ds(lr, _L)][0] == 0.0)
                def _():
                    def fz(v, c4):
                        plsc.store_scatter(
                            out_v, [jnp.full((_L,), r, jnp.int32),
                                    base3 + v * (_K * _L)], zeros)
                        return c4
                    lax.fori_loop(0, _D // _L, fz, 0)

                @pl.when(mn_v[pl.ds(lr, _L)][0] == 0.0)
                def _():
                    def fz(v, c4):
                        plsc.store_scatter(
                            out_v, [jnp.full((_L,), r, jnp.int32),
                                    base3 + v * (_K * _L) + 2], zeros)
                        return c4
                    lax.fori_loop(0, _D // _L, fz, 0)
                return cc
            lax.fori_loop(0, _C, fixr, 0)

            pltpu.sync_copy(out_v, out_hbm.at[pl.ds(pl.multiple_of(s, 8), _C)])
        return c
    lax.fori_loop(0, _QMAX, chunk, 0)


def kernel(x, lengths):
    csum = jnp.cumsum(lengths.astype(jnp.int32))
    mesh = plsc.VectorSubcoreMesh(core_axis_name="c", subcore_axis_name="s")
    run = pl.kernel(
        _sc_body,
        mesh=mesh,
        compiler_params=pltpu.CompilerParams(needs_layout_passes=False),
        out_type=jax.ShapeDtypeStruct((_N, _DK), jnp.float32),
        scratch_types=[
            pltpu.VMEM((_INROWS, _D), jnp.float32),
            pltpu.VMEM((_C, _DK), jnp.float32),
            pltpu.VMEM((256,), jnp.int32),
            pltpu.VMEM((_FMAX + 2 * _L,), jnp.float32),
            pltpu.VMEM((_FMAX + 2 * _L,), jnp.float32),
            pltpu.VMEM((_NG * _L,), jnp.int32),
            pltpu.VMEM((_NG * _L,), jnp.int32),
        ],
    )
    return run(x, csum)


# speedup vs baseline: 2.2350x; 1.0105x over previous
import jax
import jax.numpy as jnp
from jax import lax
from jax.experimental import pallas as pl
from jax.experimental.pallas import tpu as pltpu
from jax.experimental.pallas import tpu_sc as plsc

_N, _D = 32640, 512
_K = 3
_DK = _D * _K
_NW = 32
_C = 32
_NCH = _N // _C
_QMAX = (_NCH + _NW - 1) // _NW
_WIN = 64
_INROWS = 96
_L = 16
_NG = _DK // _L
_FMAX = _QMAX * _C


def _sc_body(x_hbm, csum_hbm, out_hbm,
             in_v, out_v, csum_v, mp_v, mn_v, rowm_v, col_v):
    wid = lax.axis_index("s") * 2 + lax.axis_index("c")

    pltpu.sync_copy(csum_hbm, csum_v)

    ones = jnp.ones((_L,), jnp.float32)
    zeros = jnp.zeros((_L,), jnp.float32)
    lane = lax.broadcasted_iota(jnp.int32, (_L,), 0)

    def init_flags(k, c):
        mp_v[pl.ds(k * _L, _L)] = ones
        mn_v[pl.ds(k * _L, _L)] = ones
        return c
    lax.fori_loop(0, (_FMAX + 2 * _L) // _L, init_flags, 0)

    for u in range(_NG):
        t = lane + u * _L
        j = lax.shift_right_logical(t * 21846, 16)
        rowm_v[pl.ds(u * _L, _L)] = t - _K * j
        col_v[pl.ds(u * _L, _L)] = j

    def bflags(k, c):
        cs = csum_v[pl.ds(k * _L, _L)]
        cid = lax.shift_right_logical(cs, 5)
        wv = cid & (_NW - 1)
        locs = (lax.shift_right_logical(cid, 5) * _C) + (cs & (_C - 1))
        okp = (wv == wid) & (cs < _N)
        plsc.store_scatter(mp_v, [jnp.clip(locs, 0, _FMAX - 1)], zeros,
                           mask=okp)
        ce = cs - 1
        cide = lax.shift_right_logical(ce, 5)
        wve = cide & (_NW - 1)
        loce = (lax.shift_right_logical(cide, 5) * _C) + (ce & (_C - 1))
        oke = (wve == wid) & (ce >= 0)
        plsc.store_scatter(mn_v, [jnp.clip(loce, 0, _FMAX - 1)], zeros,
                           mask=oke)
        return c
    lax.fori_loop(0, 256 // _L, bflags, 0)

    @pl.when(wid == 0)
    def _():
        plsc.store_scatter(mp_v, [lane], zeros, mask=(lane == 0))

        def zhalo(v, c):
            in_v[31, pl.ds(v * _L, _L)] = zeros
            return c
        lax.fori_loop(0, _D // _L, zhalo, 0)

    def chunk(q, c):
        cid = wid + _NW * q

        @pl.when(cid < _NCH)
        def _():
            s = cid * _C
            src_lo = pl.multiple_of(jnp.clip(s - 16, 0, _N - _WIN), 8)
            dst_lo = pl.multiple_of(src_lo - s + 32, 8)
            pltpu.sync_copy(x_hbm.at[pl.ds(src_lo, _WIN)],
                            in_v.at[pl.ds(dst_lo, _WIN)])

            @plsc.parallel_loop(0, _NG, 1, unroll=4)
            def _(u):
                ob = u * _L
                rowm = rowm_v[pl.ds(ob, _L)]
                col = col_v[pl.ds(ob, _L)]
                for r in range(_C):
                    g = plsc.load_gather(in_v, [rowm + (r + 31), col])
                    out_v[r, pl.ds(ob, _L)] = g

            def fixr(r, cc):
                lr = q * _C + r
                base3 = lane * _K

                @pl.when(mp_v[pl.ds(lr, _L)][0] == 0.0)
                def _():
                    def fz(v, c4):
                        plsc.store_scatter(
                            out_v, [jnp.full((_L,), r, jnp.int32),
                                    base3 + v * (_K * _L)], zeros)
                        return c4
                    lax.fori_loop(0, _D // _L, fz, 0)

                @pl.when(mn_v[pl.ds(lr, _L)][0] == 0.0)
                def _():
                    def fz(v, c4):
                        plsc.store_scatter(
                            out_v, [jnp.full((_L,), r, jnp.int32),
                                    base3 + v * (_K * _L) + 2], zeros)
                        return c4
                    lax.fori_loop(0, _D // _L, fz, 0)
                return cc
            lax.fori_loop(0, _C, fixr, 0)

            pltpu.sync_copy(out_v, out_hbm.at[pl.ds(pl.multiple_of(s, 8), _C)])
        return c
    lax.fori_loop(0, _QMAX, chunk, 0)


def kernel(x, lengths):
    csum = jnp.cumsum(lengths.astype(jnp.int32))
    mesh = plsc.VectorSubcoreMesh(core_axis_name="c", subcore_axis_name="s")
    run = pl.kernel(
        _sc_body,
        mesh=mesh,
        compiler_params=pltpu.CompilerParams(needs_layout_passes=False),
        out_type=jax.ShapeDtypeStruct((_N, _DK), jnp.float32),
        scratch_types=[
            pltpu.VMEM((_INROWS, _D), jnp.float32),
            pltpu.VMEM((_C, _DK), jnp.float32),
            pltpu.VMEM((256,), jnp.int32),
            pltpu.VMEM((_FMAX + 2 * _L,), jnp.float32),
            pltpu.VMEM((_FMAX + 2 * _L,), jnp.float32),
            pltpu.VMEM((_NG * _L,), jnp.int32),
            pltpu.VMEM((_NG * _L,), jnp.int32),
        ],
    )
    return run(x, csum)


# speedup vs baseline: 2.2989x; 1.0286x over previous
import jax
import jax.numpy as jnp
from jax import lax
from jax.experimental import pallas as pl
from jax.experimental.pallas import tpu as pltpu
from jax.experimental.pallas import tpu_sc as plsc

_N, _D = 32640, 512
_K = 3
_DK = _D * _K
_NW = 32
_C = 32
_NCH = _N // _C
_QMAX = (_NCH + _NW - 1) // _NW
_WIN = 48
_INROWS = 64
_L = 16
_NG = _DK // _L
_FMAX = _QMAX * _C


def _sc_body(x_hbm, csum_hbm, out_hbm,
             in_v, out_v, csum_v, mp_v, mn_v, rowm_v, col_v):
    wid = lax.axis_index("s") * 2 + lax.axis_index("c")

    pltpu.sync_copy(csum_hbm, csum_v)

    ones = jnp.ones((_L,), jnp.float32)
    zeros = jnp.zeros((_L,), jnp.float32)
    lane = lax.broadcasted_iota(jnp.int32, (_L,), 0)

    def init_flags(k, c):
        mp_v[pl.ds(k * _L, _L)] = ones
        mn_v[pl.ds(k * _L, _L)] = ones
        return c
    lax.fori_loop(0, (_FMAX + 2 * _L) // _L, init_flags, 0)

    for u in range(_NG):
        t = lane + u * _L
        j = lax.shift_right_logical(t * 21846, 16)
        rowm_v[pl.ds(u * _L, _L)] = t - _K * j
        col_v[pl.ds(u * _L, _L)] = j

    def bflags(k, c):
        cs = csum_v[pl.ds(k * _L, _L)]
        cid = lax.shift_right_logical(cs, 5)
        wv = cid & (_NW - 1)
        locs = (lax.shift_right_logical(cid, 5) * _C) + (cs & (_C - 1))
        okp = (wv == wid) & (cs < _N)
        plsc.store_scatter(mp_v, [jnp.clip(locs, 0, _FMAX - 1)], zeros,
                           mask=okp)
        ce = cs - 1
        cide = lax.shift_right_logical(ce, 5)
        wve = cide & (_NW - 1)
        loce = (lax.shift_right_logical(cide, 5) * _C) + (ce & (_C - 1))
        oke = (wve == wid) & (ce >= 0)
        plsc.store_scatter(mn_v, [jnp.clip(loce, 0, _FMAX - 1)], zeros,
                           mask=oke)
        return c
    lax.fori_loop(0, 256 // _L, bflags, 0)

    @pl.when(wid == 0)
    def _():
        plsc.store_scatter(mp_v, [lane], zeros, mask=(lane == 0))

        def zhalo(v, c):
            in_v[15, pl.ds(v * _L, _L)] = zeros
            return c
        lax.fori_loop(0, _D // _L, zhalo, 0)

    def chunk(q, c):
        cid = wid + _NW * q

        @pl.when(cid < _NCH)
        def _():
            s = cid * _C
            src_lo = pl.multiple_of(jnp.clip(s - 8, 0, _N - _WIN), 8)
            dst_lo = pl.multiple_of(src_lo - s + 16, 8)
            pltpu.sync_copy(x_hbm.at[pl.ds(src_lo, _WIN)],
                            in_v.at[pl.ds(dst_lo, _WIN)])

            @plsc.parallel_loop(0, _NG, 1, unroll=4)
            def _(u):
                ob = u * _L
                rowm = rowm_v[pl.ds(ob, _L)]
                col = col_v[pl.ds(ob, _L)]
                for r in range(_C):
                    g = plsc.load_gather(in_v, [rowm + (r + 15), col])
                    out_v[r, pl.ds(ob, _L)] = g

            def fixr(r, cc):
                lr = q * _C + r
                base3 = lane * _K

                @pl.when(mp_v[pl.ds(lr, _L)][0] == 0.0)
                def _():
                    def fz(v, c4):
                        plsc.store_scatter(
                            out_v, [jnp.full((_L,), r, jnp.int32),
                                    base3 + v * (_K * _L)], zeros)
                        return c4
                    lax.fori_loop(0, _D // _L, fz, 0)

                @pl.when(mn_v[pl.ds(lr, _L)][0] == 0.0)
                def _():
                    def fz(v, c4):
                        plsc.store_scatter(
                            out_v, [jnp.full((_L,), r, jnp.int32),
                                    base3 + v * (_K * _L) + 2], zeros)
                        return c4
                    lax.fori_loop(0, _D // _L, fz, 0)
                return cc
            lax.fori_loop(0, _C, fixr, 0)

            pltpu.sync_copy(out_v, out_hbm.at[pl.ds(pl.multiple_of(s, 8), _C)])
        return c
    lax.fori_loop(0, _QMAX, chunk, 0)


def kernel(x, lengths):
    csum = jnp.cumsum(lengths.astype(jnp.int32))
    mesh = plsc.VectorSubcoreMesh(core_axis_name="c", subcore_axis_name="s")
    run = pl.kernel(
        _sc_body,
        mesh=mesh,
        compiler_params=pltpu.CompilerParams(needs_layout_passes=False),
        out_type=jax.ShapeDtypeStruct((_N, _DK), jnp.float32),
        scratch_types=[
            pltpu.VMEM((_INROWS, _D), jnp.float32),
            pltpu.VMEM((_C, _DK), jnp.float32),
            pltpu.VMEM((256,), jnp.int32),
            pltpu.VMEM((_FMAX + 2 * _L,), jnp.float32),
            pltpu.VMEM((_FMAX + 2 * _L,), jnp.float32),
            pltpu.VMEM((_NG * _L,), jnp.int32),
            pltpu.VMEM((_NG * _L,), jnp.int32),
        ],
    )
    return run(x, csum)
